# baseline (speedup 1.0000x reference)
import jax
import jax.numpy as jnp
from jax import lax
from jax.experimental import pallas as pl
from jax.experimental.pallas import tpu as pltpu
from jax.experimental.pallas import tpu_sc as plsc

NU = 50000
NT = 10000
H = 128
NC = 2
NS = 16
NW = NC * NS
K = 128
ACC_T = 10368
DUMP = ACC_T - 1
ZROWS = ACC_T // NS
NT_PAD = 10240
E_TT_W = 5120
E_UT_W = 10240
KT = 128
R6 = 12672
NU_OUT = 4 * R6
RACC = R6 + 128
DUMP_R = RACC - 1
ZROWS_R = RACC // NS
E_TU_T = 20096

_f32 = jnp.float32


def _make_sc_tags(with_counts):
    outs = [jax.ShapeDtypeStruct((NC, NT_PAD, H), _f32),
            jax.ShapeDtypeStruct((NC, NT_PAD, H), _f32)]
    if with_counts:
        outs += [jax.ShapeDtypeStruct((NC, NT_PAD, H), _f32),
                 jax.ShapeDtypeStruct((NC, NT_PAD, H), _f32)]
    scratch = [
        pltpu.VMEM_SHARED((ACC_T, H), _f32),
        pltpu.VMEM((KT,), jnp.int32),
        pltpu.VMEM((KT,), jnp.int32),
        pltpu.VMEM((KT, H), _f32),
        pltpu.SemaphoreType.DMA,
        pltpu.VMEM((KT,), jnp.int32),
        pltpu.VMEM((KT,), jnp.int32),
        pltpu.VMEM((KT, H), _f32),
        pltpu.SemaphoreType.DMA,
    ]
    mesh = plsc.VectorSubcoreMesh(core_axis_name="c", subcore_axis_name="s")

    def body(t_h, u_h, tts, ttd, uts, utd, zf, ones_h, *refs):
        if with_counts:
            ott, out_ut, ctt, cut = refs[:4]
            refs = refs[4:]
        else:
            ott, out_ut = refs[:2]
            ctt = cut = None
            refs = refs[2:]
        acc, gidx, sidx, rows, sem, gidx2, sidx2, rows2, sem2 = refs
        cid = lax.axis_index("c")
        sid = lax.axis_index("s")
        w = cid * NS + sid

        def flush(out_f):
            plsc.subcore_barrier()
            r = NT_PAD // NS
            pltpu.sync_copy(acc.at[pl.ds(sid * r, r)],
                            out_f.at[cid, pl.ds(sid * r, r)])
            plsc.subcore_barrier()

        def feat_pass(table, src3, dst3, n_e, out_f):
            pltpu.sync_copy(zf, acc.at[pl.ds(sid * ZROWS, ZROWS)])
            plsc.subcore_barrier()

            def pair(i, carry):
                c0 = 2 * i
                pltpu.sync_copy(src3.at[w, 0, pl.ds(c0 * KT, KT)], gidx)
                pltpu.sync_copy(dst3.at[w, 0, pl.ds(c0 * KT, KT)], sidx)
                pltpu.async_copy(table.at[gidx], rows, sem)
                pltpu.sync_copy(src3.at[w, 0, pl.ds((c0 + 1) * KT, KT)],
                                gidx2)
                pltpu.sync_copy(dst3.at[w, 0, pl.ds((c0 + 1) * KT, KT)],
                                sidx2)
                pltpu.async_copy(table.at[gidx2], rows2, sem2)
                pltpu.make_async_copy(table.at[gidx], rows, sem).wait()
                pltpu.sync_copy(rows, acc.at[sidx], add=True)
                pltpu.make_async_copy(table.at[gidx2], rows2, sem2).wait()
                pltpu.sync_copy(rows2, acc.at[sidx2], add=True)
                return carry

            lax.fori_loop(0, n_e // KT // 2, pair, 0)
            flush(out_f)

        def count_pass(dst3, n_e, out_c):
            pltpu.sync_copy(zf, acc.at[pl.ds(sid * ZROWS, ZROWS)])
            pltpu.sync_copy(ones_h, rows)
            plsc.subcore_barrier()

            def step(i, carry):
                pltpu.sync_copy(dst3.at[w, 0, pl.ds(i * KT, KT)], sidx)
                pltpu.sync_copy(rows, acc.at[sidx], add=True)
                return carry

            lax.fori_loop(0, n_e // KT, step, 0)
            flush(out_c)

        feat_pass(t_h, tts, ttd, E_TT_W, ott)
        if with_counts:
            count_pass(ttd, E_TT_W, ctt)
        feat_pass(u_h, uts, utd, E_UT_W, out_ut)
        if with_counts:
            count_pass(utd, E_UT_W, cut)

    return pl.kernel(body, out_type=tuple(outs), mesh=mesh,
                     scratch_types=scratch)


def _make_sc_users():
    mesh = plsc.VectorSubcoreMesh(core_axis_name="c", subcore_axis_name="s")
    scratch = [
        pltpu.VMEM_SHARED((RACC, H), _f32),
        pltpu.VMEM((K,), jnp.int32),
        pltpu.VMEM((K,), jnp.int32),
        pltpu.VMEM((K,), jnp.int32),
        pltpu.VMEM((K,), jnp.int32),
        pltpu.VMEM((K, H), _f32),
        pltpu.SemaphoreType.DMA,
    ]

    def body(t_h, tus3, tud3, zfr, otu, acc, tdst, tsrc, gidx, sidx, rows,
             sem):
        cid = lax.axis_index("c")
        sid = lax.axis_index("s")
        for qq in range(2):
            lo = (cid * 2 + qq) * R6
            pltpu.sync_copy(zfr, acc.at[pl.ds(sid * ZROWS_R, ZROWS_R)])
            plsc.subcore_barrier()

            def step(i, carry):
                pltpu.sync_copy(tus3.at[sid, 0, pl.ds(i * K, K)], tsrc)
                pltpu.sync_copy(tud3.at[sid, 0, pl.ds(i * K, K)], tdst)
                for j in range(K // 16):
                    d = tdst[pl.ds(j * 16, 16)]
                    s = tsrc[pl.ds(j * 16, 16)]
                    inq = (d >= lo) & (d < lo + R6)
                    dump = R6 + j * 16 + lax.iota(jnp.int32, 16)
                    sidx[pl.ds(j * 16, 16)] = jnp.where(inq, d - lo, dump)
                    gidx[pl.ds(j * 16, 16)] = jnp.where(inq, s, 0)
                pltpu.async_copy(t_h.at[gidx], rows, sem).wait()
                pltpu.sync_copy(rows, acc.at[sidx], add=True)
                return carry

            lax.fori_loop(0, E_TU_T // K, step, 0)
            plsc.subcore_barrier()
            r = R6 // NS
            pltpu.sync_copy(acc.at[pl.ds(sid * r, r)],
                            otu.at[pl.ds(lo + sid * r, r)])
            plsc.subcore_barrier()

    return pl.kernel(body,
                     out_type=jax.ShapeDtypeStruct((NU_OUT, H), _f32),
                     mesh=mesh, scratch_types=scratch)


def _make_sc_ucount():
    mesh = plsc.VectorSubcoreMesh(core_axis_name="c", subcore_axis_name="s")
    scratch = [
        pltpu.VMEM_SHARED((RACC, H), _f32),
        pltpu.VMEM((K,), jnp.int32),
        pltpu.VMEM((K,), jnp.int32),
        pltpu.VMEM((K, H), _f32),
        pltpu.SemaphoreType.DMA,
    ]

    def body(tud3, zfr, ones_h, ctu, acc, tdst, sidx, onesv, sem):
        cid = lax.axis_index("c")
        sid = lax.axis_index("s")
        pltpu.sync_copy(ones_h, onesv)
        for qq in range(2):
            lo = (cid * 2 + qq) * R6
            pltpu.sync_copy(zfr, acc.at[pl.ds(sid * ZROWS_R, ZROWS_R)])
            plsc.subcore_barrier()

            def step(i, carry):
                pltpu.sync_copy(tud3.at[sid, 0, pl.ds(i * K, K)], tdst)
                for j in range(K // 16):
                    d = tdst[pl.ds(j * 16, 16)]
                    inq = (d >= lo) & (d < lo + R6)
                    dump = R6 + j * 16 + lax.iota(jnp.int32, 16)
                    sidx[pl.ds(j * 16, 16)] = jnp.where(inq, d - lo, dump)
                pltpu.sync_copy(onesv, acc.at[sidx], add=True)
                return carry

            lax.fori_loop(0, E_TU_T // K, step, 0)
            plsc.subcore_barrier()
            r = R6 // NS
            pltpu.sync_copy(acc.at[pl.ds(sid * r, r)],
                            ctu.at[pl.ds(lo + sid * r, r)])
            plsc.subcore_barrier()

    return pl.kernel(body,
                     out_type=jax.ShapeDtypeStruct((NU_OUT, H), _f32),
                     mesh=mesh, scratch_types=scratch)


def _full(i):
    return (0, 0)


def _rows(i):
    return (i, 0)


def _ln_proj(x, g, b, w, bp, br):
    n = x.shape[0]

    def tc_body(x_ref, g_ref, b_ref, w_ref, bp_ref, o_ref):
        xb = x_ref[...]
        m = jnp.mean(xb, axis=1, keepdims=True)
        xc = xb - m
        v = jnp.mean(xc * xc, axis=1, keepdims=True)
        xn = xc * lax.rsqrt(v + 1e-5) * g_ref[...] + b_ref[...]
        o_ref[...] = (jnp.dot(xn, w_ref[...], preferred_element_type=_f32)
                      + bp_ref[...])

    return pl.pallas_call(
        tc_body,
        grid=(n // br,),
        in_specs=[pl.BlockSpec((br, H), _rows),
                  pl.BlockSpec((1, H), _full), pl.BlockSpec((1, H), _full),
                  pl.BlockSpec((H, H), _full), pl.BlockSpec((1, H), _full)],
        out_specs=pl.BlockSpec((br, H), _rows),
        out_shape=jax.ShapeDtypeStruct((n, H), _f32),
    )(x, g.reshape(1, H), b.reshape(1, H), w, bp.reshape(1, H))


def _tc_tags(ott, out_ut, ctt, cut, t, wl_tt, wr_tt, wl_ut, wr_ut, bsum):
    br = 2000

    def tc_body(p0, p1, q0, q1, c0, c1, c2, c3, t_ref, wl1, wr1, wl2, wr2,
                bs, o_ref, s_ref):
        dtt = jnp.maximum(c0[...][:, 0:1] + c1[...][:, 0:1], 1.0)
        dut = jnp.maximum(c2[...][:, 0:1] + c3[...][:, 0:1], 1.0)
        a_tt = (p0[...] + p1[...]) / dtt
        a_ut = (q0[...] + q1[...]) / dut
        tb = t_ref[...]
        o = (jnp.dot(a_tt, wl1[...], preferred_element_type=_f32)
             + jnp.dot(tb, wr1[...], preferred_element_type=_f32)
             + jnp.dot(a_ut, wl2[...], preferred_element_type=_f32)
             + jnp.dot(tb, wr2[...], preferred_element_type=_f32)
             + bs[...]) * 0.5
        tn = jnp.maximum(o, 0.0)
        o_ref[...] = tn

        @pl.when(pl.program_id(0) == 0)
        def _():
            s_ref[...] = jnp.zeros_like(s_ref)

        s_ref[...] += jnp.sum(tn, axis=0, keepdims=True)

    fspec = pl.BlockSpec((br, H), _rows)
    cspec = pl.BlockSpec((br, H), _rows)
    wspec = pl.BlockSpec((H, H), _full)
    return pl.pallas_call(
        tc_body,
        grid=(NT // br,),
        in_specs=[fspec, fspec, fspec, fspec, cspec, cspec, cspec, cspec,
                  fspec, wspec, wspec, wspec, wspec,
                  pl.BlockSpec((1, H), _full)],
        out_specs=(fspec, pl.BlockSpec((1, H), _full)),
        out_shape=(jax.ShapeDtypeStruct((NT, H), _f32),
                   jax.ShapeDtypeStruct((1, H), _f32)),
    )(ott[0], ott[1], out_ut[0], out_ut[1], ctt[0], ctt[1], cut[0], cut[1],
      t, wl_tt, wr_tt, wl_ut, wr_ut, bsum.reshape(1, H))


def _tc_users(otu, ctu, u, wl, wr, bl):
    br = 2000

    def tc_body(p0, c0, u_ref, wl_ref, wr_ref, b_ref, o_ref, s_ref):
        d = jnp.maximum(c0[...][:, 0:1], 1.0)
        a = p0[...] / d
        o = (jnp.dot(a, wl_ref[...], preferred_element_type=_f32)
             + jnp.dot(u_ref[...], wr_ref[...], preferred_element_type=_f32)
             + b_ref[...])
        un = jnp.maximum(o, 0.0)
        o_ref[...] = un

        @pl.when(pl.program_id(0) == 0)
        def _():
            s_ref[...] = jnp.zeros_like(s_ref)

        s_ref[...] += jnp.sum(un, axis=0, keepdims=True)

    fspec = pl.BlockSpec((br, H), _rows)
    cspec = pl.BlockSpec((br, H), _rows)
    wspec = pl.BlockSpec((H, H), _full)
    return pl.pallas_call(
        tc_body,
        grid=(NU // br,),
        in_specs=[fspec, cspec, fspec, wspec, wspec,
                  pl.BlockSpec((1, H), _full)],
        out_specs=(fspec, pl.BlockSpec((1, H), _full)),
        out_shape=(jax.ShapeDtypeStruct((NU, H), _f32),
                   jax.ShapeDtypeStruct((1, H), _f32)),
    )(otu, ctu, u, wl, wr, bl.reshape(1, H))


def _tc_head(usum, tsum, w1, b1, w2, b2, wh, bh):
    def tc_body(us, ts, w1r, b1r, w2r, b2r, whr, bhr, o_ref):
        fr = jnp.concatenate([us[...] * (1.0 / NU), ts[...] * (1.0 / NT)],
                             axis=1)
        hh = jnp.maximum(jnp.dot(fr, w1r[...], preferred_element_type=_f32)
                         + b1r[...], 0.0)
        f2 = (jnp.dot(hh, w2r[...], preferred_element_type=_f32) + b2r[...])
        o_ref[...] = (jnp.dot(f2, whr[...], preferred_element_type=_f32)
                      + bhr[...])

    return pl.pallas_call(
        tc_body,
        out_shape=jax.ShapeDtypeStruct((1, H), _f32),
    )(usum, tsum, w1, b1.reshape(1, 2 * H), w2, b2.reshape(1, 2 * H), wh, bh)


def kernel(x_user, x_tag, params, tt_src, tt_dst, ut_src, ut_dst, tu_src,
           tu_dst):
    p = params

    def pad_edges(src, dst, nrows, padlen, dump):
        s2 = src.reshape(nrows, 1, -1)
        d2 = dst.reshape(nrows, 1, -1)
        w = padlen - s2.shape[2]
        dumps = jnp.broadcast_to(
            dump + (jnp.arange(w, dtype=jnp.int32) % 128)[None, None, :],
            (nrows, 1, w))
        return (jnp.pad(s2, ((0, 0), (0, 0), (0, w))),
                jnp.concatenate([d2, dumps], axis=2))

    tts, ttd = pad_edges(tt_src, tt_dst, NW, E_TT_W, NT_PAD)
    uts, utd = pad_edges(ut_src, ut_dst, NW, E_UT_W, NT_PAD)
    tus16, tud16 = pad_edges(tu_src, tu_dst, NS, E_TU_T, NU_OUT)
    zf = jnp.zeros((ZROWS, H), _f32)
    zfr = jnp.zeros((ZROWS_R, H), _f32)
    ones_h = jnp.ones((K, H), _f32)

    u = _ln_proj(x_user, p['ln_u_g'], p['ln_u_b'], p['proj_u_W'],
                 p['proj_u_b'], 2000)
    t = _ln_proj(x_tag, p['ln_t_g'], p['ln_t_b'], p['proj_t_W'],
                 p['proj_t_b'], 2000)

    sc_tags_c = _make_sc_tags(True)
    sc_tags = _make_sc_tags(False)
    sc_users = _make_sc_users()
    sc_ucount = _make_sc_ucount()

    ctt = cut = ctu = None
    usum = tsum = None
    for i in range(3):
        if i == 0:
            ctu = sc_ucount(tud16, zfr, ones_h)
            ott, out_ut, ctt, cut = sc_tags_c(t, u, tts, ttd, uts, utd,
                                              zf, ones_h)
        else:
            ott, out_ut = sc_tags(t, u, tts, ttd, uts, utd, zf, ones_h)
        otu = sc_users(t, tus16, tud16, zfr)
        u, usum = _tc_users(otu, ctu, u, p['c%d_tu_Wl' % i],
                            p['c%d_tu_Wr' % i], p['c%d_tu_bl' % i])
        t, tsum = _tc_tags(ott, out_ut, ctt, cut, t,
                           p['c%d_tt_Wl' % i], p['c%d_tt_Wr' % i],
                           p['c%d_ut_Wl' % i], p['c%d_ut_Wr' % i],
                           p['c%d_tt_bl' % i] + p['c%d_ut_bl' % i])

    wh = jnp.pad(jnp.concatenate([p['qpd_W'], p['ans_W'], p['ret_W']],
                                 axis=1), ((0, 0), (0, H - 3)))
    bh = jnp.pad(jnp.concatenate([p['qpd_b'], p['ans_b'], p['ret_b']]),
                 (0, H - 3)).reshape(1, H)
    out = _tc_head(usum, tsum, p['mlp_W1'], p['mlp_b1'], p['mlp_W2'],
                   p['mlp_b2'], wh, bh)
    return out[0, :3]


# speedup vs baseline: 17.9623x; 17.9623x over previous
import jax
import jax.numpy as jnp
from jax import lax
from jax.experimental import pallas as pl
from jax.experimental.pallas import tpu as pltpu
from jax.experimental.pallas import tpu_sc as plsc

NU = 50000
NT = 10000
H = 128
NC = 2
NS = 16
NW = NC * NS
K = 128
ACC_T = 10368
DUMP = ACC_T - 1
ZROWS = ACC_T // NS
NT_PAD = 10240
E_TT_W = 5120
E_UT_W = 10240
KT = 128
R6 = 12672
NU_OUT = 4 * R6
RACC = R6 + 128
DUMP_R = RACC - 1
ZROWS_R = RACC // NS
E_TU_T = 20096

_f32 = jnp.float32


def _make_sc_tags(with_counts):
    outs = [jax.ShapeDtypeStruct((NC, NT_PAD, H), _f32),
            jax.ShapeDtypeStruct((NC, NT_PAD, H), _f32)]
    if with_counts:
        outs += [jax.ShapeDtypeStruct((NC, NT_PAD, H), _f32),
                 jax.ShapeDtypeStruct((NC, NT_PAD, H), _f32)]
    scratch = [
        pltpu.VMEM_SHARED((ACC_T, H), _f32),
        pltpu.VMEM((KT,), jnp.int32),
        pltpu.VMEM((KT,), jnp.int32),
        pltpu.VMEM((KT, H), _f32),
        pltpu.SemaphoreType.DMA,
        pltpu.VMEM((KT,), jnp.int32),
        pltpu.VMEM((KT,), jnp.int32),
        pltpu.VMEM((KT, H), _f32),
        pltpu.SemaphoreType.DMA,
    ]
    mesh = plsc.VectorSubcoreMesh(core_axis_name="c", subcore_axis_name="s")

    def body(t_h, u_h, tts, ttd, uts, utd, zf, ones_h, *refs):
        if with_counts:
            ott, out_ut, ctt, cut = refs[:4]
            refs = refs[4:]
        else:
            ott, out_ut = refs[:2]
            ctt = cut = None
            refs = refs[2:]
        acc, gidx, sidx, rows, sem, gidx2, sidx2, rows2, sem2 = refs
        cid = lax.axis_index("c")
        sid = lax.axis_index("s")
        w = cid * NS + sid

        def flush(out_f):
            plsc.subcore_barrier()
            r = NT_PAD // NS
            pltpu.sync_copy(acc.at[pl.ds(sid * r, r)],
                            out_f.at[cid, pl.ds(sid * r, r)])
            plsc.subcore_barrier()

        def feat_pass(table, src3, dst3, n_e, out_f):
            pltpu.sync_copy(zf, acc.at[pl.ds(sid * ZROWS, ZROWS)])
            plsc.subcore_barrier()

            def pair(i, carry):
                c0 = 2 * i
                pltpu.sync_copy(src3.at[w, 0, pl.ds(c0 * KT, KT)], gidx)
                pltpu.sync_copy(dst3.at[w, 0, pl.ds(c0 * KT, KT)], sidx)
                pltpu.async_copy(table.at[gidx], rows, sem)
                pltpu.sync_copy(src3.at[w, 0, pl.ds((c0 + 1) * KT, KT)],
                                gidx2)
                pltpu.sync_copy(dst3.at[w, 0, pl.ds((c0 + 1) * KT, KT)],
                                sidx2)
                pltpu.async_copy(table.at[gidx2], rows2, sem2)
                pltpu.make_async_copy(table.at[gidx], rows, sem).wait()
                pltpu.sync_copy(rows, acc.at[sidx], add=True)
                pltpu.make_async_copy(table.at[gidx2], rows2, sem2).wait()
                pltpu.sync_copy(rows2, acc.at[sidx2], add=True)
                return carry

            lax.fori_loop(0, n_e // KT // 2, pair, 0)
            flush(out_f)

        def count_pass(dst3, n_e, out_c):
            pltpu.sync_copy(zf, acc.at[pl.ds(sid * ZROWS, ZROWS)])
            pltpu.sync_copy(ones_h, rows)
            plsc.subcore_barrier()

            def step(i, carry):
                pltpu.sync_copy(dst3.at[w, 0, pl.ds(i * KT, KT)], sidx)
                pltpu.sync_copy(rows, acc.at[sidx], add=True)
                return carry

            lax.fori_loop(0, n_e // KT, step, 0)
            flush(out_c)

        feat_pass(t_h, tts, ttd, E_TT_W, ott)
        if with_counts:
            count_pass(ttd, E_TT_W, ctt)
        feat_pass(u_h, uts, utd, E_UT_W, out_ut)
        if with_counts:
            count_pass(utd, E_UT_W, cut)

    return pl.kernel(body, out_type=tuple(outs), mesh=mesh,
                     scratch_types=scratch)


def _make_sc_users():
    mesh = plsc.VectorSubcoreMesh(core_axis_name="c", subcore_axis_name="s")
    scratch = [
        pltpu.VMEM_SHARED((RACC, H), _f32),
        pltpu.VMEM((K,), jnp.int32),
        pltpu.VMEM((K,), jnp.int32),
        pltpu.VMEM((K,), jnp.int32),
        pltpu.VMEM((K, H), _f32),
        pltpu.SemaphoreType.DMA,
    ]

    def body(t_h, tus3, tud3, zfr, otu, acc, tdst, gidx, sidx, rows, sem):
        cid = lax.axis_index("c")
        sid = lax.axis_index("s")
        for qq in range(2):
            lo = (cid * 2 + qq) * R6
            pltpu.sync_copy(zfr, acc.at[pl.ds(sid * ZROWS_R, ZROWS_R)])
            plsc.subcore_barrier()

            def step(i, carry):
                pltpu.sync_copy(tus3.at[sid, 0, pl.ds(i * K, K)], gidx)
                pltpu.sync_copy(tud3.at[sid, 0, pl.ds(i * K, K)], tdst)
                for j in range(K // 16):
                    d = tdst[pl.ds(j * 16, 16)]
                    inq = (d >= lo) & (d < lo + R6)
                    dump = R6 + j * 16 + lax.iota(jnp.int32, 16)
                    sidx[pl.ds(j * 16, 16)] = jnp.where(inq, d - lo, dump)
                pltpu.async_copy(t_h.at[gidx], rows, sem).wait()
                pltpu.sync_copy(rows, acc.at[sidx], add=True)
                return carry

            lax.fori_loop(0, E_TU_T // K, step, 0)
            plsc.subcore_barrier()
            r = R6 // NS
            pltpu.sync_copy(acc.at[pl.ds(sid * r, r)],
                            otu.at[pl.ds(lo + sid * r, r)])
            plsc.subcore_barrier()

    return pl.kernel(body,
                     out_type=jax.ShapeDtypeStruct((NU_OUT, H), _f32),
                     mesh=mesh, scratch_types=scratch)


def _make_sc_ucount():
    mesh = plsc.VectorSubcoreMesh(core_axis_name="c", subcore_axis_name="s")
    scratch = [
        pltpu.VMEM_SHARED((RACC, H), _f32),
        pltpu.VMEM((K,), jnp.int32),
        pltpu.VMEM((K,), jnp.int32),
        pltpu.VMEM((K, H), _f32),
        pltpu.SemaphoreType.DMA,
    ]

    def body(tud3, zfr, ones_h, ctu, acc, tdst, sidx, onesv, sem):
        cid = lax.axis_index("c")
        sid = lax.axis_index("s")
        pltpu.sync_copy(ones_h, onesv)
        for qq in range(2):
            lo = (cid * 2 + qq) * R6
            pltpu.sync_copy(zfr, acc.at[pl.ds(sid * ZROWS_R, ZROWS_R)])
            plsc.subcore_barrier()

            def step(i, carry):
                pltpu.sync_copy(tud3.at[sid, 0, pl.ds(i * K, K)], tdst)
                for j in range(K // 16):
                    d = tdst[pl.ds(j * 16, 16)]
                    inq = (d >= lo) & (d < lo + R6)
                    dump = R6 + j * 16 + lax.iota(jnp.int32, 16)
                    sidx[pl.ds(j * 16, 16)] = jnp.where(inq, d - lo, dump)
                pltpu.sync_copy(onesv, acc.at[sidx], add=True)
                return carry

            lax.fori_loop(0, E_TU_T // K, step, 0)
            plsc.subcore_barrier()
            r = R6 // NS
            pltpu.sync_copy(acc.at[pl.ds(sid * r, r)],
                            ctu.at[pl.ds(lo + sid * r, r)])
            plsc.subcore_barrier()

    return pl.kernel(body,
                     out_type=jax.ShapeDtypeStruct((NU_OUT, H), _f32),
                     mesh=mesh, scratch_types=scratch)


def _full(i):
    return (0, 0)


def _rows(i):
    return (i, 0)


def _ln_proj(x, g, b, w, bp, br):
    n = x.shape[0]

    def tc_body(x_ref, g_ref, b_ref, w_ref, bp_ref, o_ref):
        xb = x_ref[...]
        m = jnp.mean(xb, axis=1, keepdims=True)
        xc = xb - m
        v = jnp.mean(xc * xc, axis=1, keepdims=True)
        xn = xc * lax.rsqrt(v + 1e-5) * g_ref[...] + b_ref[...]
        o_ref[...] = (jnp.dot(xn, w_ref[...], preferred_element_type=_f32)
                      + bp_ref[...])

    return pl.pallas_call(
        tc_body,
        grid=(n // br,),
        in_specs=[pl.BlockSpec((br, H), _rows),
                  pl.BlockSpec((1, H), _full), pl.BlockSpec((1, H), _full),
                  pl.BlockSpec((H, H), _full), pl.BlockSpec((1, H), _full)],
        out_specs=pl.BlockSpec((br, H), _rows),
        out_shape=jax.ShapeDtypeStruct((n, H), _f32),
    )(x, g.reshape(1, H), b.reshape(1, H), w, bp.reshape(1, H))


def _tc_tags(ott, out_ut, ctt, cut, t, wl_tt, wr_tt, wl_ut, wr_ut, bsum):
    br = 2000

    def tc_body(p0, p1, q0, q1, c0, c1, c2, c3, t_ref, wl1, wr1, wl2, wr2,
                bs, o_ref, s_ref):
        dtt = jnp.maximum(c0[...][:, 0:1] + c1[...][:, 0:1], 1.0)
        dut = jnp.maximum(c2[...][:, 0:1] + c3[...][:, 0:1], 1.0)
        a_tt = (p0[...] + p1[...]) / dtt
        a_ut = (q0[...] + q1[...]) / dut
        tb = t_ref[...]
        o = (jnp.dot(a_tt, wl1[...], preferred_element_type=_f32)
             + jnp.dot(tb, wr1[...], preferred_element_type=_f32)
             + jnp.dot(a_ut, wl2[...], preferred_element_type=_f32)
             + jnp.dot(tb, wr2[...], preferred_element_type=_f32)
             + bs[...]) * 0.5
        tn = jnp.maximum(o, 0.0)
        o_ref[...] = tn

        @pl.when(pl.program_id(0) == 0)
        def _():
            s_ref[...] = jnp.zeros_like(s_ref)

        s_ref[...] += jnp.sum(tn, axis=0, keepdims=True)

    fspec = pl.BlockSpec((br, H), _rows)
    cspec = pl.BlockSpec((br, H), _rows)
    wspec = pl.BlockSpec((H, H), _full)
    return pl.pallas_call(
        tc_body,
        grid=(NT // br,),
        in_specs=[fspec, fspec, fspec, fspec, cspec, cspec, cspec, cspec,
                  fspec, wspec, wspec, wspec, wspec,
                  pl.BlockSpec((1, H), _full)],
        out_specs=(fspec, pl.BlockSpec((1, H), _full)),
        out_shape=(jax.ShapeDtypeStruct((NT, H), _f32),
                   jax.ShapeDtypeStruct((1, H), _f32)),
    )(ott[0], ott[1], out_ut[0], out_ut[1], ctt[0], ctt[1], cut[0], cut[1],
      t, wl_tt, wr_tt, wl_ut, wr_ut, bsum.reshape(1, H))


def _tc_users(otu, ctu, u, wl, wr, bl):
    br = 2000

    def tc_body(p0, c0, u_ref, wl_ref, wr_ref, b_ref, o_ref, s_ref):
        d = jnp.maximum(c0[...][:, 0:1], 1.0)
        a = p0[...] / d
        o = (jnp.dot(a, wl_ref[...], preferred_element_type=_f32)
             + jnp.dot(u_ref[...], wr_ref[...], preferred_element_type=_f32)
             + b_ref[...])
        un = jnp.maximum(o, 0.0)
        o_ref[...] = un

        @pl.when(pl.program_id(0) == 0)
        def _():
            s_ref[...] = jnp.zeros_like(s_ref)

        s_ref[...] += jnp.sum(un, axis=0, keepdims=True)

    fspec = pl.BlockSpec((br, H), _rows)
    cspec = pl.BlockSpec((br, H), _rows)
    wspec = pl.BlockSpec((H, H), _full)
    return pl.pallas_call(
        tc_body,
        grid=(NU // br,),
        in_specs=[fspec, cspec, fspec, wspec, wspec,
                  pl.BlockSpec((1, H), _full)],
        out_specs=(fspec, pl.BlockSpec((1, H), _full)),
        out_shape=(jax.ShapeDtypeStruct((NU, H), _f32),
                   jax.ShapeDtypeStruct((1, H), _f32)),
    )(otu, ctu, u, wl, wr, bl.reshape(1, H))


def _tc_head(usum, tsum, w1, b1, w2, b2, wh, bh):
    def tc_body(us, ts, w1r, b1r, w2r, b2r, whr, bhr, o_ref):
        fr = jnp.concatenate([us[...] * (1.0 / NU), ts[...] * (1.0 / NT)],
                             axis=1)
        hh = jnp.maximum(jnp.dot(fr, w1r[...], preferred_element_type=_f32)
                         + b1r[...], 0.0)
        f2 = (jnp.dot(hh, w2r[...], preferred_element_type=_f32) + b2r[...])
        o_ref[...] = (jnp.dot(f2, whr[...], preferred_element_type=_f32)
                      + bhr[...])

    return pl.pallas_call(
        tc_body,
        out_shape=jax.ShapeDtypeStruct((1, H), _f32),
    )(usum, tsum, w1, b1.reshape(1, 2 * H), w2, b2.reshape(1, 2 * H), wh, bh)


def kernel(x_user, x_tag, params, tt_src, tt_dst, ut_src, ut_dst, tu_src,
           tu_dst):
    p = params

    def pad_edges(src, dst, nrows, padlen, dump):
        s2 = src.reshape(nrows, 1, -1)
        d2 = dst.reshape(nrows, 1, -1)
        w = padlen - s2.shape[2]
        dumps = jnp.broadcast_to(
            dump + (jnp.arange(w, dtype=jnp.int32) % 128)[None, None, :],
            (nrows, 1, w))
        return (jnp.pad(s2, ((0, 0), (0, 0), (0, w))),
                jnp.concatenate([d2, dumps], axis=2))

    tts, ttd = pad_edges(tt_src, tt_dst, NW, E_TT_W, NT_PAD)
    uts, utd = pad_edges(ut_src, ut_dst, NW, E_UT_W, NT_PAD)
    tus16, tud16 = pad_edges(tu_src, tu_dst, NS, E_TU_T, NU_OUT)
    zf = jnp.zeros((ZROWS, H), _f32)
    zfr = jnp.zeros((ZROWS_R, H), _f32)
    ones_h = jnp.ones((K, H), _f32)

    u = _ln_proj(x_user, p['ln_u_g'], p['ln_u_b'], p['proj_u_W'],
                 p['proj_u_b'], 2000)
    t = _ln_proj(x_tag, p['ln_t_g'], p['ln_t_b'], p['proj_t_W'],
                 p['proj_t_b'], 2000)

    sc_tags_c = _make_sc_tags(True)
    sc_tags = _make_sc_tags(False)
    sc_users = _make_sc_users()
    sc_ucount = _make_sc_ucount()

    ctt = cut = ctu = None
    usum = tsum = None
    for i in range(3):
        if i == 0:
            ctu = sc_ucount(tud16, zfr, ones_h)
            ott, out_ut, ctt, cut = sc_tags_c(t, u, tts, ttd, uts, utd,
                                              zf, ones_h)
        else:
            ott, out_ut = sc_tags(t, u, tts, ttd, uts, utd, zf, ones_h)
        otu = sc_users(t, tus16, tud16, zfr)
        u, usum = _tc_users(otu, ctu, u, p['c%d_tu_Wl' % i],
                            p['c%d_tu_Wr' % i], p['c%d_tu_bl' % i])
        t, tsum = _tc_tags(ott, out_ut, ctt, cut, t,
                           p['c%d_tt_Wl' % i], p['c%d_tt_Wr' % i],
                           p['c%d_ut_Wl' % i], p['c%d_ut_Wr' % i],
                           p['c%d_tt_bl' % i] + p['c%d_ut_bl' % i])

    wh = jnp.pad(jnp.concatenate([p['qpd_W'], p['ans_W'], p['ret_W']],
                                 axis=1), ((0, 0), (0, H - 3)))
    bh = jnp.pad(jnp.concatenate([p['qpd_b'], p['ans_b'], p['ret_b']]),
                 (0, H - 3)).reshape(1, H)
    out = _tc_head(usum, tsum, p['mlp_W1'], p['mlp_b1'], p['mlp_W2'],
                   p['mlp_b2'], wh, bh)
    return out[0, :3]


# speedup vs baseline: 17.9653x; 1.0002x over previous
import jax
import jax.numpy as jnp
from jax import lax
from jax.experimental import pallas as pl
from jax.experimental.pallas import tpu as pltpu
from jax.experimental.pallas import tpu_sc as plsc

NU = 50000
NT = 10000
H = 128
NC = 2
NS = 16
NW = NC * NS
K = 128
ACC_T = 10368
DUMP = ACC_T - 1
ZROWS = ACC_T // NS
NT_PAD = 10240
E_TT_W = 5120
E_UT_W = 10240
KT = 128
R6 = 12672
NU_OUT = 4 * R6
RACC = R6 + 128
DUMP_R = RACC - 1
ZROWS_R = RACC // NS
E_TU_T = 20096

_f32 = jnp.float32


def _make_sc_tags(with_counts):
    outs = [jax.ShapeDtypeStruct((NC, NT_PAD, H), _f32),
            jax.ShapeDtypeStruct((NC, NT_PAD, H), _f32)]
    if with_counts:
        outs += [jax.ShapeDtypeStruct((NC, NT_PAD, H), _f32),
                 jax.ShapeDtypeStruct((NC, NT_PAD, H), _f32)]
    scratch = [
        pltpu.VMEM_SHARED((ACC_T, H), _f32),
        pltpu.VMEM((KT,), jnp.int32),
        pltpu.VMEM((KT,), jnp.int32),
        pltpu.VMEM((KT, H), _f32),
        pltpu.SemaphoreType.DMA,
        pltpu.VMEM((KT,), jnp.int32),
        pltpu.VMEM((KT,), jnp.int32),
        pltpu.VMEM((KT, H), _f32),
        pltpu.SemaphoreType.DMA,
    ]
    mesh = plsc.VectorSubcoreMesh(core_axis_name="c", subcore_axis_name="s")

    def body(t_h, u_h, tts, ttd, uts, utd, zf, ones_h, *refs):
        if with_counts:
            ott, out_ut, ctt, cut = refs[:4]
            refs = refs[4:]
        else:
            ott, out_ut = refs[:2]
            ctt = cut = None
            refs = refs[2:]
        acc, gidx, sidx, rows, sem, gidx2, sidx2, rows2, sem2 = refs
        cid = lax.axis_index("c")
        sid = lax.axis_index("s")
        w = cid * NS + sid

        def flush(out_f):
            plsc.subcore_barrier()
            r = NT_PAD // NS
            pltpu.sync_copy(acc.at[pl.ds(sid * r, r)],
                            out_f.at[cid, pl.ds(sid * r, r)])
            plsc.subcore_barrier()

        def feat_pass(table, src3, dst3, n_e, out_f):
            pltpu.sync_copy(zf, acc.at[pl.ds(sid * ZROWS, ZROWS)])
            plsc.subcore_barrier()

            def pair(i, carry):
                c0 = 2 * i
                pltpu.sync_copy(src3.at[w, 0, pl.ds(c0 * KT, KT)], gidx)
                pltpu.sync_copy(dst3.at[w, 0, pl.ds(c0 * KT, KT)], sidx)
                pltpu.async_copy(table.at[gidx], rows, sem)
                pltpu.sync_copy(src3.at[w, 0, pl.ds((c0 + 1) * KT, KT)],
                                gidx2)
                pltpu.sync_copy(dst3.at[w, 0, pl.ds((c0 + 1) * KT, KT)],
                                sidx2)
                pltpu.async_copy(table.at[gidx2], rows2, sem2)
                pltpu.make_async_copy(table.at[gidx], rows, sem).wait()
                pltpu.sync_copy(rows, acc.at[sidx], add=True)
                pltpu.make_async_copy(table.at[gidx2], rows2, sem2).wait()
                pltpu.sync_copy(rows2, acc.at[sidx2], add=True)
                return carry

            lax.fori_loop(0, n_e // KT // 2, pair, 0)
            flush(out_f)

        def count_pass(dst3, n_e, out_c):
            pltpu.sync_copy(zf, acc.at[pl.ds(sid * ZROWS, ZROWS)])
            pltpu.sync_copy(ones_h, rows)
            plsc.subcore_barrier()

            def step(i, carry):
                pltpu.sync_copy(dst3.at[w, 0, pl.ds(i * KT, KT)], sidx)
                pltpu.sync_copy(rows, acc.at[sidx], add=True)
                return carry

            lax.fori_loop(0, n_e // KT, step, 0)
            flush(out_c)

        feat_pass(t_h, tts, ttd, E_TT_W, ott)
        if with_counts:
            count_pass(ttd, E_TT_W, ctt)
        feat_pass(u_h, uts, utd, E_UT_W, out_ut)
        if with_counts:
            count_pass(utd, E_UT_W, cut)

    return pl.kernel(body, out_type=tuple(outs), mesh=mesh,
                     scratch_types=scratch)


def _make_sc_users():
    mesh = plsc.VectorSubcoreMesh(core_axis_name="c", subcore_axis_name="s")
    scratch = [
        pltpu.VMEM_SHARED((RACC, H), _f32),
        pltpu.VMEM((K,), jnp.int32),
        pltpu.VMEM((K,), jnp.int32),
        pltpu.VMEM((K,), jnp.int32),
        pltpu.VMEM((K, H), _f32),
        pltpu.SemaphoreType.DMA,
    ]

    def body(t_h, tus3, tud3, zfr, otu, acc, tdst, gidx, sidx, rows, sem):
        cid = lax.axis_index("c")
        sid = lax.axis_index("s")
        for qq in range(2):
            lo = (cid * 2 + qq) * R6
            pltpu.sync_copy(zfr, acc.at[pl.ds(sid * ZROWS_R, ZROWS_R)])
            plsc.subcore_barrier()

            def step(i, carry):
                pltpu.sync_copy(tus3.at[sid, 0, pl.ds(i * K, K)], gidx)
                pltpu.sync_copy(tud3.at[sid, 0, pl.ds(i * K, K)], tdst)
                for j in range(K // 16):
                    d = tdst[pl.ds(j * 16, 16)]
                    inq = (d >= lo) & (d < lo + R6)
                    dump = R6 + j * 16 + lax.iota(jnp.int32, 16)
                    sidx[pl.ds(j * 16, 16)] = jnp.where(inq, d - lo, dump)
                pltpu.async_copy(t_h.at[gidx], rows, sem).wait()
                pltpu.sync_copy(rows, acc.at[sidx], add=True)
                return carry

            lax.fori_loop(0, E_TU_T // K, step, 0)
            plsc.subcore_barrier()
            r = R6 // NS
            pltpu.sync_copy(acc.at[pl.ds(sid * r, r)],
                            otu.at[pl.ds(lo + sid * r, r)])
            plsc.subcore_barrier()

    return pl.kernel(body,
                     out_type=jax.ShapeDtypeStruct((NU_OUT, H), _f32),
                     mesh=mesh, scratch_types=scratch)


def _make_sc_ucount():
    mesh = plsc.VectorSubcoreMesh(core_axis_name="c", subcore_axis_name="s")
    scratch = [
        pltpu.VMEM_SHARED((RACC, H), _f32),
        pltpu.VMEM((K,), jnp.int32),
        pltpu.VMEM((K,), jnp.int32),
        pltpu.VMEM((K, H), _f32),
        pltpu.SemaphoreType.DMA,
    ]

    def body(tud3, zfr, ones_h, ctu, acc, tdst, sidx, onesv, sem):
        cid = lax.axis_index("c")
        sid = lax.axis_index("s")
        pltpu.sync_copy(ones_h, onesv)
        for qq in range(2):
            lo = (cid * 2 + qq) * R6
            pltpu.sync_copy(zfr, acc.at[pl.ds(sid * ZROWS_R, ZROWS_R)])
            plsc.subcore_barrier()

            def step(i, carry):
                pltpu.sync_copy(tud3.at[sid, 0, pl.ds(i * K, K)], tdst)
                for j in range(K // 16):
                    d = tdst[pl.ds(j * 16, 16)]
                    inq = (d >= lo) & (d < lo + R6)
                    dump = R6 + j * 16 + lax.iota(jnp.int32, 16)
                    sidx[pl.ds(j * 16, 16)] = jnp.where(inq, d - lo, dump)
                pltpu.sync_copy(onesv, acc.at[sidx], add=True)
                return carry

            lax.fori_loop(0, E_TU_T // K, step, 0)
            plsc.subcore_barrier()
            r = R6 // NS
            pltpu.sync_copy(acc.at[pl.ds(sid * r, r)],
                            ctu.at[pl.ds(lo + sid * r, r)])
            plsc.subcore_barrier()

    return pl.kernel(body,
                     out_type=jax.ShapeDtypeStruct((NU_OUT, H), _f32),
                     mesh=mesh, scratch_types=scratch)


def _full(i):
    return (0, 0)


def _rows(i):
    return (i, 0)


def _ln_proj(x, g, b, w, bp, br):
    n = x.shape[0]

    def tc_body(x_ref, g_ref, b_ref, w_ref, bp_ref, o_ref):
        xb = x_ref[...]
        m = jnp.mean(xb, axis=1, keepdims=True)
        xc = xb - m
        v = jnp.mean(xc * xc, axis=1, keepdims=True)
        xn = xc * lax.rsqrt(v + 1e-5) * g_ref[...] + b_ref[...]
        o_ref[...] = (jnp.dot(xn, w_ref[...], preferred_element_type=_f32)
                      + bp_ref[...])

    return pl.pallas_call(
        tc_body,
        grid=(n // br,),
        in_specs=[pl.BlockSpec((br, H), _rows),
                  pl.BlockSpec((1, H), _full), pl.BlockSpec((1, H), _full),
                  pl.BlockSpec((H, H), _full), pl.BlockSpec((1, H), _full)],
        out_specs=pl.BlockSpec((br, H), _rows),
        out_shape=jax.ShapeDtypeStruct((n, H), _f32),
    )(x, g.reshape(1, H), b.reshape(1, H), w, bp.reshape(1, H))


def _tc_tags(ott, out_ut, ctt, cut, t, wl_tt, wr_tt, wl_ut, wr_ut, bsum):
    br = 2000

    def tc_body(p0, p1, q0, q1, c0, c1, c2, c3, t_ref, wl1, wr1, wl2, wr2,
                bs, o_ref, s_ref):
        dtt = jnp.maximum(c0[...][:, 0:1] + c1[...][:, 0:1], 1.0)
        dut = jnp.maximum(c2[...][:, 0:1] + c3[...][:, 0:1], 1.0)
        a_tt = (p0[...] + p1[...]) / dtt
        a_ut = (q0[...] + q1[...]) / dut
        tb = t_ref[...]
        o = (jnp.dot(a_tt, wl1[...], preferred_element_type=_f32)
             + jnp.dot(tb, wr1[...], preferred_element_type=_f32)
             + jnp.dot(a_ut, wl2[...], preferred_element_type=_f32)
             + jnp.dot(tb, wr2[...], preferred_element_type=_f32)
             + bs[...]) * 0.5
        tn = jnp.maximum(o, 0.0)
        o_ref[...] = tn

        @pl.when(pl.program_id(0) == 0)
        def _():
            s_ref[...] = jnp.zeros_like(s_ref)

        s_ref[...] += jnp.sum(tn, axis=0, keepdims=True)

    fspec = pl.BlockSpec((br, H), _rows)
    cspec = pl.BlockSpec((br, H), _rows)
    wspec = pl.BlockSpec((H, H), _full)
    return pl.pallas_call(
        tc_body,
        grid=(NT // br,),
        in_specs=[fspec, fspec, fspec, fspec, cspec, cspec, cspec, cspec,
                  fspec, wspec, wspec, wspec, wspec,
                  pl.BlockSpec((1, H), _full)],
        out_specs=(fspec, pl.BlockSpec((1, H), _full)),
        out_shape=(jax.ShapeDtypeStruct((NT, H), _f32),
                   jax.ShapeDtypeStruct((1, H), _f32)),
    )(ott[0], ott[1], out_ut[0], out_ut[1], ctt[0], ctt[1], cut[0], cut[1],
      t, wl_tt, wr_tt, wl_ut, wr_ut, bsum.reshape(1, H))


def _tc_users(otu, ctu, u, wl, wr, bl):
    br = 2000

    def tc_body(p0, c0, u_ref, wl_ref, wr_ref, b_ref, o_ref, s_ref):
        d = jnp.maximum(c0[...][:, 0:1], 1.0)
        a = p0[...] / d
        o = (jnp.dot(a, wl_ref[...], preferred_element_type=_f32)
             + jnp.dot(u_ref[...], wr_ref[...], preferred_element_type=_f32)
             + b_ref[...])
        un = jnp.maximum(o, 0.0)
        o_ref[...] = un

        @pl.when(pl.program_id(0) == 0)
        def _():
            s_ref[...] = jnp.zeros_like(s_ref)

        s_ref[...] += jnp.sum(un, axis=0, keepdims=True)

    fspec = pl.BlockSpec((br, H), _rows)
    cspec = pl.BlockSpec((br, H), _rows)
    wspec = pl.BlockSpec((H, H), _full)
    return pl.pallas_call(
        tc_body,
        grid=(NU // br,),
        in_specs=[fspec, cspec, fspec, wspec, wspec,
                  pl.BlockSpec((1, H), _full)],
        out_specs=(fspec, pl.BlockSpec((1, H), _full)),
        out_shape=(jax.ShapeDtypeStruct((NU, H), _f32),
                   jax.ShapeDtypeStruct((1, H), _f32)),
    )(otu, ctu, u, wl, wr, bl.reshape(1, H))


def _tc_head(usum, tsum, w1, b1, w2, b2, wh, bh):
    def tc_body(us, ts, w1r, b1r, w2r, b2r, whr, bhr, o_ref):
        fr = jnp.concatenate([us[...] * (1.0 / NU), ts[...] * (1.0 / NT)],
                             axis=1)
        hh = jnp.maximum(jnp.dot(fr, w1r[...], preferred_element_type=_f32)
                         + b1r[...], 0.0)
        f2 = (jnp.dot(hh, w2r[...], preferred_element_type=_f32) + b2r[...])
        o_ref[...] = (jnp.dot(f2, whr[...], preferred_element_type=_f32)
                      + bhr[...])

    return pl.pallas_call(
        tc_body,
        out_shape=jax.ShapeDtypeStruct((1, H), _f32),
    )(usum, tsum, w1, b1.reshape(1, 2 * H), w2, b2.reshape(1, 2 * H), wh, bh)


def kernel(x_user, x_tag, params, tt_src, tt_dst, ut_src, ut_dst, tu_src,
           tu_dst):
    p = params

    def pad_edges(src, dst, nrows, padlen, dump):
        s2 = src.reshape(nrows, 1, -1)
        d2 = dst.reshape(nrows, 1, -1)
        w = padlen - s2.shape[2]
        dumps = jnp.broadcast_to(
            dump + (jnp.arange(w, dtype=jnp.int32) % 128)[None, None, :],
            (nrows, 1, w))
        return (jnp.pad(s2, ((0, 0), (0, 0), (0, w))),
                jnp.concatenate([d2, dumps], axis=2))

    tts, ttd = pad_edges(tt_src, tt_dst, NW, E_TT_W, NT_PAD)
    uts, utd = pad_edges(ut_src, ut_dst, NW, E_UT_W, NT_PAD)
    tus16, tud16 = pad_edges(tu_src, tu_dst, NS, E_TU_T, NU_OUT)
    zf = jnp.zeros((ZROWS, H), _f32)
    zfr = jnp.zeros((ZROWS_R, H), _f32)
    ones_h = jnp.ones((K, H), _f32)

    u = _ln_proj(x_user, p['ln_u_g'], p['ln_u_b'], p['proj_u_W'],
                 p['proj_u_b'], 2000)
    t = _ln_proj(x_tag, p['ln_t_g'], p['ln_t_b'], p['proj_t_W'],
                 p['proj_t_b'], 2000)

    sc_tags_c = _make_sc_tags(True)
    sc_tags = _make_sc_tags(False)
    sc_users = _make_sc_users()
    sc_ucount = _make_sc_ucount()

    ctt = cut = ctu = None
    usum = tsum = None
    for i in range(3):
        if i == 0:
            ctu = sc_ucount(tud16, zfr, ones_h)
        otu = sc_users(t, tus16, tud16, zfr)
        if i == 0:
            ott, out_ut, ctt, cut = sc_tags_c(t, u, tts, ttd, uts, utd,
                                              zf, ones_h)
        else:
            ott, out_ut = sc_tags(t, u, tts, ttd, uts, utd, zf, ones_h)
        u, usum = _tc_users(otu, ctu, u, p['c%d_tu_Wl' % i],
                            p['c%d_tu_Wr' % i], p['c%d_tu_bl' % i])
        t, tsum = _tc_tags(ott, out_ut, ctt, cut, t,
                           p['c%d_tt_Wl' % i], p['c%d_tt_Wr' % i],
                           p['c%d_ut_Wl' % i], p['c%d_ut_Wr' % i],
                           p['c%d_tt_bl' % i] + p['c%d_ut_bl' % i])

    wh = jnp.pad(jnp.concatenate([p['qpd_W'], p['ans_W'], p['ret_W']],
                                 axis=1), ((0, 0), (0, H - 3)))
    bh = jnp.pad(jnp.concatenate([p['qpd_b'], p['ans_b'], p['ret_b']]),
                 (0, H - 3)).reshape(1, H)
    out = _tc_head(usum, tsum, p['mlp_W1'], p['mlp_b1'], p['mlp_W2'],
                   p['mlp_b2'], wh, bh)
    return out[0, :3]
